# baseline (speedup 1.0000x reference)
import functools

import jax
import jax.numpy as jnp
import numpy as np
from jax import lax
from jax.experimental import pallas as pl
from jax.experimental.pallas import tpu as pltpu
from jax.experimental.pallas import tpu_sc as plsc

_G = 128
_L = 16
_NB = 5


def _embed_flat(table, idx, offs, n, e):
    info = plsc.get_sparse_core_info()
    nw = info.num_cores * info.num_subcores
    rpw = n // nw
    nwin = rpw // _G
    assert n == nw * nwin * _G

    mesh = plsc.VectorSubcoreMesh(core_axis_name="core",
                                  subcore_axis_name="subcore")

    @functools.partial(
        pl.kernel,
        out_type=jax.ShapeDtypeStruct((n, e), table.dtype),
        mesh=mesh,
        scratch_types=(
            [pltpu.VMEM((rpw,), jnp.int32),
             pltpu.VMEM((rpw,), jnp.int32)]
            + [pltpu.VMEM((_G,), jnp.int32) for _ in range(_NB)]
            + [pltpu.VMEM((_G, e), jnp.float32) for _ in range(_NB)]
            + [pltpu.SemaphoreType.DMA for _ in range(2 * _NB)]
        ),
    )
    def run(table_hbm, idx_hbm, offs_hbm, out_hbm, idx_v, offs_v, *bufs):
        fx = bufs[:_NB]
        rows = bufs[_NB:2 * _NB]
        gsem = bufs[2 * _NB:3 * _NB]
        osem = bufs[3 * _NB:4 * _NB]

        wid = (lax.axis_index("subcore") * info.num_cores
               + lax.axis_index("core"))
        base = wid * rpw

        pre = _NB * _G
        head = [
            pltpu.async_copy(idx_hbm.at[pl.ds(base, pre)],
                             idx_v.at[pl.ds(0, pre)], gsem[0]),
            pltpu.async_copy(offs_hbm.at[pl.ds(base, pre)],
                             offs_v.at[pl.ds(0, pre)], gsem[1]),
        ]
        tail = [
            pltpu.async_copy(idx_hbm.at[pl.ds(base + pre, rpw - pre)],
                             idx_v.at[pl.ds(pre, rpw - pre)], osem[0]),
            pltpu.async_copy(offs_hbm.at[pl.ds(base + pre, rpw - pre)],
                             offs_v.at[pl.ds(pre, rpw - pre)], osem[1]),
        ]
        for h in head:
            h.wait()

        def form_and_fire(j, b):
            for c in range(0, _G, _L):
                src = pl.ds(j * _G + c, _L)
                fx[b].at[pl.ds(c, _L)][...] = (
                    idx_v.at[src][...] + offs_v.at[src][...])
            pltpu.async_copy(table_hbm.at[fx[b]], rows[b], gsem[b])

        def wait_gather(b):
            pltpu.make_async_copy(table_hbm.at[fx[b]], rows[b],
                                  gsem[b]).wait()

        def start_out(i, b):
            pltpu.async_copy(rows[b],
                             out_hbm.at[pl.ds(base + i * _G, _G)], osem[b])

        def wait_out(i, b):
            pltpu.make_async_copy(rows[b],
                                  out_hbm.at[pl.ds(base + i * _G, _G)],
                                  osem[b]).wait()

        for w in range(_NB - 1):
            form_and_fire(w, w)
        for t in tail:
            t.wait()

        @pl.loop(0, nwin // _NB)
        def _(o):
            for b in range(_NB):
                i = o * _NB + b
                j = i + _NB - 1
                bj = (b + _NB - 1) % _NB
                @pl.when(j < nwin)
                def _():
                    @pl.when(j >= _NB)
                    def _():
                        wait_out(j - _NB, bj)
                    form_and_fire(j, bj)

                wait_gather(b)
                start_out(i, b)

        for w in range(nwin - _NB, nwin):
            wait_out(w, w % _NB)

    return run(table, idx, offs)


def kernel(x, parameter):
    s, p, e = parameter.shape
    b = x.shape[0]
    n = b * s
    table = parameter.reshape(s * p, e)
    idx = x.reshape(n).astype(jnp.int32)
    offs = jnp.asarray(np.tile(np.arange(s, dtype=np.int32) * p, b))
    out = _embed_flat(table, idx, offs, n, e)
    return out.reshape(b, s, e)


# speedup vs baseline: 1.0015x; 1.0015x over previous
import functools

import jax
import jax.numpy as jnp
import numpy as np
from jax import lax
from jax.experimental import pallas as pl
from jax.experimental.pallas import tpu as pltpu
from jax.experimental.pallas import tpu_sc as plsc

_G = 128
_L = 16
_NB = 5


def _embed_flat(table, idx, offs, n, e):
    info = plsc.get_sparse_core_info()
    nw = info.num_cores * info.num_subcores
    rpw = n // nw
    nwin = rpw // _G
    assert n == nw * nwin * _G

    mesh = plsc.VectorSubcoreMesh(core_axis_name="core",
                                  subcore_axis_name="subcore")

    @functools.partial(
        pl.kernel,
        out_type=jax.ShapeDtypeStruct((n, e), table.dtype),
        mesh=mesh,
        scratch_types=(
            [pltpu.VMEM((rpw,), jnp.int32),
             pltpu.VMEM((rpw,), jnp.int32)]
            + [pltpu.VMEM((_G,), jnp.int32) for _ in range(_NB)]
            + [pltpu.VMEM((_G, e), jnp.float32) for _ in range(_NB)]
            + [pltpu.SemaphoreType.DMA for _ in range(2 * _NB)]
        ),
    )
    def run(table_hbm, idx_hbm, offs_hbm, out_hbm, idx_v, offs_v, *bufs):
        fx = bufs[:_NB]
        rows = bufs[_NB:2 * _NB]
        gsem = bufs[2 * _NB:3 * _NB]
        osem = bufs[3 * _NB:4 * _NB]

        wid = (lax.axis_index("subcore") * info.num_cores
               + lax.axis_index("core"))
        base = wid * rpw

        pre = _NB * _G
        head = [
            pltpu.async_copy(idx_hbm.at[pl.ds(base, pre)],
                             idx_v.at[pl.ds(0, pre)], gsem[0]),
            pltpu.async_copy(offs_hbm.at[pl.ds(base, pre)],
                             offs_v.at[pl.ds(0, pre)], gsem[1]),
        ]
        tail = [
            pltpu.async_copy(idx_hbm.at[pl.ds(base + pre, rpw - pre)],
                             idx_v.at[pl.ds(pre, rpw - pre)], osem[0]),
            pltpu.async_copy(offs_hbm.at[pl.ds(base + pre, rpw - pre)],
                             offs_v.at[pl.ds(pre, rpw - pre)], osem[1]),
        ]
        for h in head:
            h.wait()

        def form_and_fire(j, b):
            @pl.loop(0, _G, step=_L)
            def _(c):
                src = pl.ds(j * _G + c, _L)
                fx[b].at[pl.ds(c, _L)][...] = (
                    idx_v.at[src][...] + offs_v.at[src][...])
            pltpu.async_copy(table_hbm.at[fx[b]], rows[b], gsem[b])

        def wait_gather(b):
            pltpu.make_async_copy(table_hbm.at[fx[b]], rows[b],
                                  gsem[b]).wait()

        def start_out(i, b):
            pltpu.async_copy(rows[b],
                             out_hbm.at[pl.ds(base + i * _G, _G)], osem[b])

        def wait_out(i, b):
            pltpu.make_async_copy(rows[b],
                                  out_hbm.at[pl.ds(base + i * _G, _G)],
                                  osem[b]).wait()

        for w in range(_NB - 1):
            form_and_fire(w, w)
        for t in tail:
            t.wait()

        @pl.loop(0, nwin // _NB)
        def _(o):
            for b in range(_NB):
                i = o * _NB + b
                j = i + _NB - 1
                bj = (b + _NB - 1) % _NB
                @pl.when(j < nwin)
                def _():
                    @pl.when(j >= _NB)
                    def _():
                        wait_out(j - _NB, bj)
                    form_and_fire(j, bj)

                wait_gather(b)
                start_out(i, b)

        for w in range(nwin - _NB, nwin):
            wait_out(w, w % _NB)

    return run(table, idx, offs)


def kernel(x, parameter):
    s, p, e = parameter.shape
    b = x.shape[0]
    n = b * s
    table = parameter.reshape(s * p, e)
    idx = x.reshape(n).astype(jnp.int32)
    offs = jnp.asarray(np.tile(np.arange(s, dtype=np.int32) * p, b))
    out = _embed_flat(table, idx, offs, n, e)
    return out.reshape(b, s, e)
